# baseline (speedup 1.0000x reference)
import functools

import jax
import jax.numpy as jnp
from jax import lax
from jax.experimental import pallas as pl
from jax.experimental.pallas import tpu as pltpu
from jax.experimental.pallas import tpu_sc as plsc

N_NODES = 10000
D_FEAT = 128
N_EDGES = 320000

NC = 2
NS = 16
L = 16
NW = NC * NS
E_W = N_EDGES // NW
B = 80
CH = E_W // B
G = B // L
K = 8
NBUF = 4


P = D_FEAT // 2


def _tile_body(xu_hbm, xm_hbm, uidx_hbm, midx_hbm, out_hbm, xu_sp,
               uidx_v, midx_v, u0, m0, u1, m1, u2, m2, u3, m3, out_v,
               sem0, sem1, sem2, sem3,
               semu0, semu1, semu2, semu3):
    sid = lax.axis_index("s")
    wid = sid * NC + lax.axis_index("c")
    base = wid * E_W

    R_T = 624
    pltpu.sync_copy(xu_hbm.at[pl.ds(sid * R_T, R_T)],
                    xu_sp.at[pl.ds(sid * R_T, R_T)])
    rem = NS * R_T

    @pl.when(sid == NS - 1)
    def _():
        pltpu.sync_copy(xu_hbm.at[pl.ds(rem, N_NODES - rem)],
                        xu_sp.at[pl.ds(rem, N_NODES - rem)])

    plsc.subcore_barrier()

    pltpu.sync_copy(uidx_hbm.at[pl.ds(base, E_W)], uidx_v)
    pltpu.sync_copy(midx_hbm.at[pl.ds(base, E_W)], midx_v)

    bufs = ((u0, m0, sem0), (u1, m1, sem1), (u2, m2, sem2), (u3, m3, sem3))

    def start(c, b):
        ub, mb, sem = bufs[b]
        pltpu.async_copy(xu_hbm.at[uidx_v.at[pl.ds(c * B, B)]], ub, sem)
        pltpu.async_copy(xm_hbm.at[midx_v.at[pl.ds(c * B, B)]], mb, sem)

    def drain(b):
        ub, mb, sem = bufs[b]
        pltpu.make_async_copy(xu_hbm.at[uidx_v.at[pl.ds(0, B)]], ub,
                              sem).wait()
        pltpu.make_async_copy(xm_hbm.at[uidx_v.at[pl.ds(0, B)]], mb,
                              sem).wait()

    def compute(c, b):
        ub, mb, _ = bufs[b]
        off = c * B
        for g in range(G):
            rows = jnp.arange(L, dtype=jnp.int32) + g * L
            zero = jnp.zeros((L,), jnp.float32)
            cols0 = jnp.arange(L, dtype=jnp.int32)

            def d_body(_, carry):
                cols, *accs = carry
                new_accs = []
                for k in range(K):
                    col = ((cols + k) if k else cols) & (D_FEAT - 1)
                    uv = plsc.load_gather(ub, [rows, col])
                    mv = plsc.load_gather(mb, [rows, col])
                    new_accs.append(accs[k] + uv * mv)
                return (cols + K, *new_accs)

            res = lax.fori_loop(0, D_FEAT // K, d_body,
                                (cols0,) + (zero,) * K)
            accs = list(res[1:])
            while len(accs) > 1:
                accs = [a + b_ for a, b_ in zip(accs[::2], accs[1::2])]
            out_v[pl.ds(off + g * L, L)] = accs[0]

    for b in range(NBUF):
        start(b, b)

    def quad_body(j, carry):
        c0 = NBUF * j
        for b in range(NBUF):
            drain(b)
            start(c0 + b + NBUF, b)
        return carry

    lax.fori_loop(0, (CH - (NBUF + 1)) // NBUF, quad_body, 0)

    c0 = ((CH - (NBUF + 1)) // NBUF) * NBUF
    drain(0)
    start(CH - 1, 0)
    for b in range(1, NBUF):
        drain(b)
    drain(0)

    pltpu.sync_copy(out_v, out_hbm.at[pl.ds(base, B)])


@functools.partial(
    pl.kernel,
    mesh=plsc.VectorSubcoreMesh(core_axis_name="c", subcore_axis_name="s"),
    out_type=jax.ShapeDtypeStruct((N_EDGES,), jnp.float32),
    compiler_params=pltpu.CompilerParams(needs_layout_passes=False,
                                         use_tc_tiling_on_sc=False),
    scratch_types=[
        pltpu.VMEM_SHARED((N_NODES, D_FEAT // 2), jnp.int32),
        pltpu.VMEM((E_W,), jnp.int32),
        pltpu.VMEM((E_W,), jnp.int32),
        pltpu.VMEM((B, D_FEAT // 2), jnp.int32),
        pltpu.VMEM((B, D_FEAT), jnp.float32),
        pltpu.VMEM((B, D_FEAT // 2), jnp.int32),
        pltpu.VMEM((B, D_FEAT), jnp.float32),
        pltpu.VMEM((B, D_FEAT // 2), jnp.int32),
        pltpu.VMEM((B, D_FEAT), jnp.float32),
        pltpu.VMEM((B, D_FEAT // 2), jnp.int32),
        pltpu.VMEM((B, D_FEAT), jnp.float32),
        pltpu.VMEM((B,), jnp.float32),
        pltpu.SemaphoreType.DMA,
        pltpu.SemaphoreType.DMA,
        pltpu.SemaphoreType.DMA,
        pltpu.SemaphoreType.DMA,
        pltpu.SemaphoreType.DMA,
        pltpu.SemaphoreType.DMA,
        pltpu.SemaphoreType.DMA,
        pltpu.SemaphoreType.DMA,
    ],
)
def _edge_dot_sc(xu_hbm, xm_hbm, uidx_hbm, midx_hbm, out_hbm, xu_sp,
                 uidx_v, midx_v, u0, m0, u1, m1, u2, m2, u3, m3, out_v,
                 sem0, sem1, sem2, sem3, semu0, semu1, semu2, semu3):
    _tile_body(xu_hbm, xm_hbm, uidx_hbm, midx_hbm, out_hbm, xu_sp,
               uidx_v, midx_v, u0, m0, u1, m1, u2, m2, u3, m3, out_v,
               sem0, sem1, sem2, sem3, semu0, semu1, semu2, semu3)


def kernel(x_user, x_movie, edge_label_index):
    idx = edge_label_index.astype(jnp.int32)
    xu_p = lax.bitcast_convert_type(
        x_user.astype(jnp.bfloat16).reshape(N_NODES, D_FEAT // 2, 2),
        jnp.int32)
    return _edge_dot_sc(xu_p, x_movie, idx[0], idx[1])


# speedup vs baseline: 1.0352x; 1.0352x over previous
import functools

import jax
import jax.numpy as jnp
from jax import lax
from jax.experimental import pallas as pl
from jax.experimental.pallas import tpu as pltpu
from jax.experimental.pallas import tpu_sc as plsc

N_NODES = 10000
D_FEAT = 128
N_EDGES = 320000

NC = 2
NS = 16
L = 16
NW = NC * NS
E_W = N_EDGES // NW
B = 80
CH = E_W // B
G = B // L
K = 8
NBUF = 4


def _tile_body(xu_hbm, xm_hbm, uidx_hbm, midx_hbm, out_hbm,
               uidx_v, midx_v, u0, m0, u1, m1, u2, m2, u3, m3, out_v,
               sem0, sem1, sem2, sem3):
    wid = lax.axis_index("s") * NC + lax.axis_index("c")
    base = wid * E_W

    ci = pltpu.async_copy(uidx_hbm.at[pl.ds(base, E_W)], uidx_v, sem0)
    cm = pltpu.async_copy(midx_hbm.at[pl.ds(base, E_W)], midx_v, sem0)
    ci.wait()
    cm.wait()

    bufs = ((u0, m0, sem0), (u1, m1, sem1), (u2, m2, sem2), (u3, m3, sem3))

    def start(c, b):
        ub, mb, sem = bufs[b]
        pltpu.async_copy(xu_hbm.at[uidx_v.at[pl.ds(c * B, B)]], ub, sem)
        pltpu.async_copy(xm_hbm.at[midx_v.at[pl.ds(c * B, B)]], mb, sem)

    def drain(b):
        ub, mb, sem = bufs[b]
        pltpu.make_async_copy(xu_hbm.at[uidx_v.at[pl.ds(0, B)]], ub,
                              sem).wait()
        pltpu.make_async_copy(xm_hbm.at[uidx_v.at[pl.ds(0, B)]], mb,
                              sem).wait()

    def compute(c, b):
        ub, mb, _ = bufs[b]
        off = c * B
        for g in range(G):
            rows = jnp.arange(L, dtype=jnp.int32) + g * L
            zero = jnp.zeros((L,), jnp.float32)
            cols0 = jnp.arange(L, dtype=jnp.int32)

            def d_body(_, carry):
                cols, *accs = carry
                new_accs = []
                for k in range(K):
                    col = ((cols + k) if k else cols) & (D_FEAT - 1)
                    uv = plsc.load_gather(ub, [rows, col])
                    mv = plsc.load_gather(mb, [rows, col])
                    new_accs.append(accs[k] + uv * mv)
                return (cols + K, *new_accs)

            res = lax.fori_loop(0, D_FEAT // K, d_body,
                                (cols0,) + (zero,) * K)
            accs = list(res[1:])
            while len(accs) > 1:
                accs = [a + b_ for a, b_ in zip(accs[::2], accs[1::2])]
            out_v[pl.ds(off + g * L, L)] = accs[0]

    for b in range(NBUF):
        start(b, b)

    def quad_body(j, carry):
        c0 = NBUF * j
        for b in range(NBUF):
            drain(b)
            compute(c0 + b, b)
            start(c0 + b + NBUF, b)
        return carry

    lax.fori_loop(0, (CH - (NBUF + 1)) // NBUF, quad_body, 0)

    c0 = ((CH - (NBUF + 1)) // NBUF) * NBUF
    drain(0)
    compute(c0, 0)
    start(CH - 1, 0)
    for b in range(1, NBUF):
        drain(b)
        compute(c0 + b, b)
    drain(0)
    compute(CH - 1, 0)

    pltpu.sync_copy(out_v, out_hbm.at[pl.ds(base, E_W)])


@functools.partial(
    pl.kernel,
    mesh=plsc.VectorSubcoreMesh(core_axis_name="c", subcore_axis_name="s"),
    out_type=jax.ShapeDtypeStruct((N_EDGES,), jnp.float32),
    compiler_params=pltpu.CompilerParams(needs_layout_passes=False),
    scratch_types=[
        pltpu.VMEM((E_W,), jnp.int32),
        pltpu.VMEM((E_W,), jnp.int32),
        pltpu.VMEM((B, D_FEAT), jnp.float32),
        pltpu.VMEM((B, D_FEAT), jnp.float32),
        pltpu.VMEM((B, D_FEAT), jnp.float32),
        pltpu.VMEM((B, D_FEAT), jnp.float32),
        pltpu.VMEM((B, D_FEAT), jnp.float32),
        pltpu.VMEM((B, D_FEAT), jnp.float32),
        pltpu.VMEM((B, D_FEAT), jnp.float32),
        pltpu.VMEM((B, D_FEAT), jnp.float32),
        pltpu.VMEM((E_W,), jnp.float32),
        pltpu.SemaphoreType.DMA,
        pltpu.SemaphoreType.DMA,
        pltpu.SemaphoreType.DMA,
        pltpu.SemaphoreType.DMA,
    ],
)
def _edge_dot_sc(xu_hbm, xm_hbm, uidx_hbm, midx_hbm, out_hbm,
                 uidx_v, midx_v, u0, m0, u1, m1, u2, m2, u3, m3, out_v,
                 sem0, sem1, sem2, sem3):
    _tile_body(xu_hbm, xm_hbm, uidx_hbm, midx_hbm, out_hbm,
               uidx_v, midx_v, u0, m0, u1, m1, u2, m2, u3, m3, out_v,
               sem0, sem1, sem2, sem3)


def kernel(x_user, x_movie, edge_label_index):
    idx = edge_label_index.astype(jnp.int32)
    return _edge_dot_sc(x_user, x_movie, idx[0], idx[1])
